# baseline (speedup 1.0000x reference)
import jax
import jax.numpy as jnp
from jax import lax
from jax.experimental import pallas as pl
from jax.experimental.pallas import tpu as pltpu
from jax.experimental.pallas import tpu_sc as plsc

_BATCH = 4096
_E = 200
_D = 64
_NC = 2
_NS = 16
_NW = _NC * _NS
_BPW = _BATCH // _NW
_G = 2
_GL = _G * _E
_NGRP = _BPW // _G
_CHUNKS = ((0, 128), (128, 72))


def _body(x_hbm, tab_hbm, vec_hbm, val_hbm,
          idxraw, safe, vbuf, buf, semg0, semg1, semo0, semo1, semv0, semv1):
    wid = lax.axis_index("s") * _NC + lax.axis_index("c")
    b0 = wid * _BPW
    semg = (semg0, semg1)
    semo = (semo0, semo1)
    semv = (semv0, semv1)

    def prep(g, p, first):
        bb = b0 + g * _G
        if not first:
            pltpu.make_async_copy(vbuf.at[p], val_hbm.at[pl.ds(0, _GL)],
                                  semv[p]).wait()
        pltpu.sync_copy(x_hbm.at[pl.ds(bb * _E, _GL)], idxraw.at[p])

        def cvec(j, c):
            o = pl.multiple_of(j * 16, 16)
            v = idxraw[p, pl.ds(o, 16)]
            valid = v != -1
            safe[p, pl.ds(o, 16)] = jnp.where(valid, v, 0)
            vbuf[p, pl.ds(o, 16)] = jnp.where(valid, 1.0, 0.0)
            return c

        lax.fori_loop(0, _GL // 16, cvec, 0)
        if not first:
            pltpu.make_async_copy(buf.at[p], vec_hbm.at[pl.ds(0, _GL)],
                                  semo[p]).wait()
        for k in range(_G):
            for (lo, ln) in _CHUNKS:
                pltpu.make_async_copy(
                    tab_hbm.at[safe.at[p, pl.ds(k * _E + lo, ln)]],
                    buf.at[p, pl.ds(k * _E + lo, ln)], semg[p]).start()

    def outcopy(g, p):
        bb = b0 + g * _G
        pltpu.make_async_copy(tab_hbm.at[pl.ds(0, _GL)], buf.at[p],
                              semg[p]).wait()
        pltpu.make_async_copy(
            buf.at[p], vec_hbm.at[pl.ds(bb * _E, _GL)], semo[p]).start()
        pltpu.make_async_copy(
            vbuf.at[p], val_hbm.at[pl.ds(bb * _E, _GL)], semv[p]).start()

    prep(0, 0, True)

    def iteration(g, first):
        prep(g + 1, 1, first)
        outcopy(g, 0)

        @pl.when(g + 2 < _NGRP)
        def _():
            prep(g + 2, 0, False)

        outcopy(g + 1, 1)

    iteration(0, True)

    def outer(u, c):
        iteration(u * 2, False)
        return c

    lax.fori_loop(1, _NGRP // 2, outer, 0)

    for p in (0, 1):
        pltpu.make_async_copy(buf.at[p], vec_hbm.at[pl.ds(0, _GL)],
                              semo[p]).wait()
        pltpu.make_async_copy(vbuf.at[p], val_hbm.at[pl.ds(0, _GL)],
                              semv[p]).wait()


def kernel(x, entity_vectors):
    mesh = plsc.VectorSubcoreMesh(core_axis_name="c", subcore_axis_name="s")
    run = pl.kernel(
        _body,
        out_type=(
            jax.ShapeDtypeStruct((_BATCH * _E, _D), jnp.float32),
            jax.ShapeDtypeStruct((_BATCH * _E,), jnp.float32),
        ),
        mesh=mesh,
        compiler_params=pltpu.CompilerParams(use_tc_tiling_on_sc=False),
        scratch_types=[
            pltpu.VMEM((2, _GL), jnp.int32),
            pltpu.VMEM((2, _GL), jnp.int32),
            pltpu.VMEM((2, _GL), jnp.float32),
            pltpu.VMEM((2, _GL, _D), jnp.float32),
            pltpu.SemaphoreType.DMA,
            pltpu.SemaphoreType.DMA,
            pltpu.SemaphoreType.DMA,
            pltpu.SemaphoreType.DMA,
            pltpu.SemaphoreType.DMA,
            pltpu.SemaphoreType.DMA,
        ],
    )
    vecs, valid = run(x.reshape(-1), entity_vectors)
    del valid
    vec4 = vecs.reshape(_BATCH, 1, _E, _D)
    maskf = (x != -1).astype(jnp.float32)
    mask4 = jnp.broadcast_to(maskf.reshape(_BATCH, 1, _E, 1), vec4.shape)
    return jnp.concatenate([vec4, mask4], axis=1)


# speedup vs baseline: 1.0026x; 1.0026x over previous
import jax
import jax.numpy as jnp
from jax import lax
from jax.experimental import pallas as pl
from jax.experimental.pallas import tpu as pltpu
from jax.experimental.pallas import tpu_sc as plsc

_BATCH = 4096
_E = 200
_D = 64
_NC = 2
_NS = 16
_NW = _NC * _NS
_BPW = _BATCH // _NW
_G = 2
_GL = _G * _E
_NGRP = _BPW // _G
_CHUNKS = ((0, 128), (128, 72))


def _body(x_hbm, tab_hbm, vec_hbm, val_hbm,
          idxraw, safe, vbuf, buf, semg0, semg1, semo0, semo1, semv0, semv1):
    wid = lax.axis_index("s") * _NC + lax.axis_index("c")
    b0 = wid * _BPW
    semg = (semg0, semg1)
    semo = (semo0, semo1)
    semv = (semv0, semv1)

    def prep(g, p, first):
        bb = b0 + g * _G
        if not first:
            pltpu.make_async_copy(vbuf.at[p], val_hbm.at[pl.ds(0, _GL)],
                                  semv[p]).wait()
        pltpu.sync_copy(x_hbm.at[pl.ds(bb * _E, _GL)], idxraw.at[p])

        def cvec(j, c):
            o = pl.multiple_of(j * 16, 16)
            v = idxraw[p, pl.ds(o, 16)]
            valid = v != -1
            safe[p, pl.ds(o, 16)] = jnp.where(valid, v, 0)
            vbuf[p, pl.ds(o, 16)] = jnp.where(valid, 1.0, 0.0)
            return c

        lax.fori_loop(0, _GL // 16, cvec, 0)
        if not first:
            pltpu.make_async_copy(buf.at[p], vec_hbm.at[pl.ds(0, _GL)],
                                  semo[p]).wait()
        for k in range(_G):
            for (lo, ln) in _CHUNKS:
                pltpu.make_async_copy(
                    tab_hbm.at[safe.at[p, pl.ds(k * _E + lo, ln)]],
                    buf.at[p, pl.ds(k * _E + lo, ln)], semg[p]).start()

    def outcopy(g, p):
        bb = b0 + g * _G
        pltpu.make_async_copy(tab_hbm.at[pl.ds(0, _GL)], buf.at[p],
                              semg[p]).wait()
        pltpu.make_async_copy(
            buf.at[p], vec_hbm.at[pl.ds(bb * _E, _GL)], semo[p]).start()
        pltpu.make_async_copy(
            vbuf.at[p], val_hbm.at[pl.ds(bb * _E, _GL)], semv[p]).start()

    prep(0, 0, True)

    def iteration(g, first):
        prep(g + 1, 1, first)
        outcopy(g, 0)

        @pl.when(g + 2 < _NGRP)
        def _():
            prep(g + 2, 0, False)

        outcopy(g + 1, 1)

    iteration(0, True)

    def outer(u, c):
        iteration(u * 2, False)
        return c

    lax.fori_loop(1, _NGRP // 2, outer, 0)

    for p in (0, 1):
        pltpu.make_async_copy(buf.at[p], vec_hbm.at[pl.ds(0, _GL)],
                              semo[p]).wait()
        pltpu.make_async_copy(vbuf.at[p], val_hbm.at[pl.ds(0, _GL)],
                              semv[p]).wait()


def kernel(x, entity_vectors):
    mesh = plsc.VectorSubcoreMesh(core_axis_name="c", subcore_axis_name="s")
    run = pl.kernel(
        _body,
        out_type=(
            jax.ShapeDtypeStruct((_BATCH * _E, _D), jnp.float32),
            jax.ShapeDtypeStruct((_BATCH * _E,), jnp.float32),
        ),
        mesh=mesh,
        compiler_params=pltpu.CompilerParams(use_tc_tiling_on_sc=False),
        scratch_types=[
            pltpu.VMEM((2, _GL), jnp.int32),
            pltpu.VMEM((2, _GL), jnp.int32),
            pltpu.VMEM((2, _GL), jnp.float32),
            pltpu.VMEM((2, _GL, _D), jnp.float32),
            pltpu.SemaphoreType.DMA,
            pltpu.SemaphoreType.DMA,
            pltpu.SemaphoreType.DMA,
            pltpu.SemaphoreType.DMA,
            pltpu.SemaphoreType.DMA,
            pltpu.SemaphoreType.DMA,
        ],
    )
    vecs, valid = run(x.reshape(-1), entity_vectors)
    vec4 = vecs.reshape(_BATCH, 1, _E, _D)
    mask4 = jnp.broadcast_to(valid.reshape(_BATCH, 1, _E, 1), vec4.shape)
    return jnp.concatenate([vec4, mask4], axis=1)
